# baseline (speedup 1.0000x reference)
import functools

import jax
import jax.numpy as jnp
from jax import lax
from jax.experimental import pallas as pl
from jax.experimental.pallas import tpu as pltpu
from jax.experimental.pallas import tpu_sc as plsc

N = 10000
NP = 10240
E = 320000
C = 128
H = 64
NSUB = 16
NCORE = 2
ROW = 128
EP = 2560
E2 = EP * ROW
ECH = EP // NSUB
NPS = NP // NSUB

_f32 = jnp.float32


@functools.lru_cache(maxsize=None)
def _make_sc_layer():
  out_type = [
      jax.ShapeDtypeStruct((NCORE, NP, H), _f32),
      jax.ShapeDtypeStruct((NCORE, NP, H), _f32),
      jax.ShapeDtypeStruct((NP,), _f32),
      jax.ShapeDtypeStruct((NP,), _f32),
  ]
  mesh = plsc.VectorSubcoreMesh(core_axis_name="c", subcore_axis_name="s",
                                num_cores=NCORE, num_subcores=NSUB)
  scratch_types = [
      pltpu.VMEM_SHARED((NP, H), _f32),
      pltpu.VMEM_SHARED((NP,), _f32),
      pltpu.VMEM((4, 3, ROW), jnp.int32),
      pltpu.VMEM((3, ROW, H), jnp.bfloat16),
      pltpu.VMEM((3, ROW, H), _f32),
      pltpu.VMEM((ROW,), _f32),
      pltpu.VMEM((NPS, H), _f32),
      pltpu.VMEM((NPS,), _f32),
      pltpu.SemaphoreType.DMA((4,)),
      pltpu.SemaphoreType.DMA((3,)),
      pltpu.SemaphoreType.DMA((3,)),
      pltpu.SemaphoreType.DMA((3,)),
  ]

  def body(xg1, xg2, comb, agg1, agg2, deg1, deg2, *refs):
    (acc_sp, deg_sp, cb, rows, msg, ones, zacc, zdeg,
     sem_i, sem_g, sem_s, sem_d) = refs
    c = lax.axis_index("c")
    s = lax.axis_index("s")
    rbase = s * NPS
    ebase = s * ECH

    @pl.loop(0, ROW // 16)
    def _(j):
      ones[pl.ds(j * 16, 16)] = jnp.ones((16,), _f32)

    @pl.loop(0, NPS // 16)
    def _(j):
      zdeg[pl.ds(j * 16, 16)] = jnp.zeros((16,), _f32)

    @pl.loop(0, NPS)
    def _(i):
      for b in range(H // 16):
        zacc[i, pl.ds(b * 16, 16)] = jnp.zeros((16,), _f32)

    def run_pass(xg, g_pl, s_pl, agg_out, deg_out, deg_core):
      pltpu.sync_copy(zacc, acc_sp.at[pl.ds(rbase, NPS)])

      @pl.when(c == deg_core)
      def _():
        pltpu.sync_copy(zdeg, deg_sp.at[pl.ds(rbase, NPS)])

      plsc.subcore_barrier()

      def idx_load(j):
        slot = lax.rem(j, 4)
        pltpu.async_copy(comb.at[ebase + j], cb.at[slot], sem_i.at[slot])

      def idx_wait(j):
        slot = lax.rem(j, 4)
        pltpu.make_async_copy(comb.at[ebase + j], cb.at[slot],
                              sem_i.at[slot]).wait()

      def prep(j):
        slot = lax.rem(j, 4)
        for q in range(ROW // 16):
          sl = pl.ds(q * 16, 16)
          cb[slot, g_pl, sl] = cb[slot, g_pl, sl] + c * NP

      def gather_issue(j):
        slot = lax.rem(j, 4)
        m = lax.rem(j, 3)
        pltpu.async_copy(xg.at[cb.at[slot, g_pl]], rows.at[m], sem_g.at[m])

      def gather_wait(j):
        slot = lax.rem(j, 4)
        m = lax.rem(j, 3)
        pltpu.make_async_copy(xg.at[cb.at[slot, g_pl]], rows.at[m],
                              sem_g.at[m]).wait()

      def scale(j):
        slot = lax.rem(j, 4)
        m = lax.rem(j, 3)
        mask = jnp.int32(-65536)

        @pl.loop(0, ROW // 16)
        def _(g):
          w16 = plsc.bitcast(cb[slot, 2, pl.ds(g * 16, 16)], _f32)
          base = g * 16
          for half in range(2):
            es = [base + half * 8 + e for e in range(8)]
            ws = [w16[half * 8 + e] for e in range(8)]
            vals = [[rows[m, i, pl.ds(b * 32, 32)] for b in range(H // 32)]
                    for i in es]
            for ei, i in enumerate(es):
              for b in range(H // 32):
                v = plsc.bitcast(vals[ei][b], jnp.int32)
                lo = plsc.bitcast(jnp.left_shift(v, 16), _f32)
                hi = plsc.bitcast(jnp.bitwise_and(v, mask), _f32)
                msg[m, i, pl.ds(b * 32, 16)] = lo * ws[ei]
                msg[m, i, pl.ds(b * 32 + 16, 16)] = hi * ws[ei]

      def scatter_issue(j):
        slot = lax.rem(j, 4)
        m = lax.rem(j, 3)
        pltpu.async_copy(msg.at[m], acc_sp.at[cb.at[slot, s_pl]],
                         sem_s.at[m], add=True)

        @pl.when(c == deg_core)
        def _():
          pltpu.async_copy(ones, deg_sp.at[cb.at[slot, s_pl]], sem_d.at[m],
                           add=True)

      def scatter_wait(j):
        slot = lax.rem(j, 4)
        m = lax.rem(j, 3)
        pltpu.make_async_copy(msg.at[m], acc_sp.at[cb.at[slot, s_pl]],
                              sem_s.at[m]).wait()

        @pl.when(c == deg_core)
        def _():
          pltpu.make_async_copy(ones, deg_sp.at[cb.at[slot, s_pl]],
                                sem_d.at[m]).wait()

      idx_load(0)
      idx_load(1)
      idx_load(2)
      idx_wait(0)
      prep(0)
      gather_issue(0)
      idx_wait(1)
      prep(1)
      gather_issue(1)

      @pl.loop(0, ECH)
      def _(j):
        j2 = j + 2

        @pl.when(j2 < ECH)
        def _():
          idx_wait(j2)
          prep(j2)

          @pl.when(j >= 1)
          def _():
            scatter_wait(j - 1)

          gather_issue(j2)

        gather_wait(j)
        scale(j)
        scatter_issue(j)

        @pl.when(j + 3 < ECH)
        def _():
          idx_load(j + 3)

      scatter_wait(ECH - 3)
      scatter_wait(ECH - 2)
      scatter_wait(ECH - 1)
      plsc.subcore_barrier()

      pltpu.sync_copy(acc_sp.at[pl.ds(rbase, NPS)],
                      agg_out.at[c, pl.ds(rbase, NPS)])

      @pl.when(c == deg_core)
      def _():
        pltpu.sync_copy(deg_sp.at[pl.ds(rbase, NPS)],
                        deg_out.at[pl.ds(rbase, NPS)])

      plsc.subcore_barrier()

    run_pass(xg1, 0, 1, agg1, deg1, 0)
    run_pass(xg2, 1, 0, agg2, deg2, 1)

  return pl.kernel(body, out_type=out_type, mesh=mesh,
                   scratch_types=scratch_types,
                   compiler_params=pltpu.CompilerParams(
                       use_tc_tiling_on_sc=False,
                       needs_layout_passes=False))


def _make_dense(relu, split_out):
  R = 1024
  grid = (NP // R,)

  def body(aggA, degA, xA, aggB, degB, xB,
           WlA, WrA, WlB, WrB, bA, bB, outA, outB):
    def half(agg, deg, x, Wl, Wr, b):
      inv = 1.0 / jnp.maximum(deg[...], 1.0)
      acc = jnp.dot(agg[0] * inv, Wl[0], preferred_element_type=_f32)
      acc += jnp.dot(agg[1] * inv, Wl[1], preferred_element_type=_f32)
      acc += jnp.dot(x[0], Wr[0], preferred_element_type=_f32)
      acc += jnp.dot(x[1], Wr[1], preferred_element_type=_f32)
      acc += b[...]
      if relu:
        acc = jnp.maximum(acc, 0.0)
      return acc

    a = half(aggA, degA, xA, WlA, WrA, bA)
    bb = half(aggB, degB, xB, WlB, WrB, bB)
    if split_out:
      outA[0], outA[1] = a[:, :H], a[:, H:]
      outB[0], outB[1] = bb[:, :H], bb[:, H:]
    else:
      outA[...] = a
      outB[...] = bb

  feat = pl.BlockSpec((NCORE, R, H), lambda i: (0, i, 0))
  deg = pl.BlockSpec((R, 1), lambda i: (i, 0))
  wspec = pl.BlockSpec((NCORE, H, C), lambda i: (0, 0, 0))
  bspec = pl.BlockSpec((1, C), lambda i: (0, 0))
  if split_out:
    out_specs = [feat, feat]
    out_shape = [jax.ShapeDtypeStruct((NCORE, NP, H), _f32)] * 2
  else:
    out_specs = [pl.BlockSpec((R, C), lambda i: (i, 0))] * 2
    out_shape = [jax.ShapeDtypeStruct((NP, C), _f32)] * 2
  return pl.pallas_call(
      body,
      grid=grid,
      in_specs=[feat, deg, feat, feat, deg, feat,
                wspec, wspec, wspec, wspec, bspec, bspec],
      out_specs=out_specs,
      out_shape=out_shape,
  )


_dense0 = _make_dense(relu=True, split_out=True)
_dense1 = _make_dense(relu=False, split_out=False)


def _sig_body(x_ref, o_ref):
  o_ref[...] = jax.nn.sigmoid(x_ref[...])


_sigmoid_tc = pl.pallas_call(
    _sig_body, out_shape=jax.ShapeDtypeStruct((EP, ROW), _f32))


def _to_halves(x):
  xp = jnp.pad(x, ((0, NP - N), (0, 0)))
  return xp.reshape(NP, NCORE, H).transpose(1, 0, 2)


import numpy as _np
_PERM64 = _np.concatenate([_np.arange(0, 32, 2), _np.arange(1, 32, 2),
                           _np.arange(32, 64, 2), _np.arange(33, 64, 2)])
_PERM128 = _np.concatenate([_PERM64, 64 + _PERM64])


@jax.jit
def kernel(s, t, edge_index, edge_weight,
           Ws_l0, Ws_r0, Wt_l0, Wt_r0,
           Ws_l1, Ws_r1, Wt_l1, Wt_r1,
           bs0, bt0, bs1, bt1):
  s2 = _to_halves(s)
  t2 = _to_halves(t)
  pad = E2 - E
  pad_idx = N + jnp.arange(pad, dtype=jnp.int32) % (NP - N)
  src = jnp.concatenate([edge_index[0], pad_idx])
  dst = jnp.concatenate([edge_index[1], pad_idx])
  ewp = jnp.concatenate([edge_weight, jnp.zeros((pad,), _f32)])
  sig = _sigmoid_tc(ewp.reshape(EP, ROW))
  comb = jnp.stack([src.reshape(EP, ROW), dst.reshape(EP, ROW),
                    jax.lax.bitcast_convert_type(sig, jnp.int32)], axis=1)

  flat = lambda x: x.reshape(NCORE * NP, H).astype(jnp.bfloat16)
  aggs0, aggt0, degd, degs = _make_sc_layer()(flat(t2), flat(s2), comb)
  degd = degd.reshape(NP, 1)
  degs = degs.reshape(NP, 1)

  wl = lambda w: w[_PERM128].reshape(NCORE, H, C)
  wr = lambda w: w.reshape(NCORE, H, C)
  s1, t1 = _dense0(aggs0, degd, t2, aggt0, degs, s2,
                   wl(Ws_l0), wr(Ws_r0), wl(Wt_l0), wr(Wt_r0),
                   bs0.reshape(1, C), bt0.reshape(1, C))

  aggs1, aggt1, _, _ = _make_sc_layer()(flat(t1), flat(s1), comb)
  s_out, t_out = _dense1(aggs1, degd, t1, aggt1, degs, s1,
                         wl(Ws_l1), wr(Ws_r1), wl(Wt_l1), wr(Wt_r1),
                         bs1.reshape(1, C), bt1.reshape(1, C))
  return (s_out[:N], t_out[:N])


# speedup vs baseline: 1.0707x; 1.0707x over previous
import functools

import jax
import jax.numpy as jnp
from jax import lax
from jax.experimental import pallas as pl
from jax.experimental.pallas import tpu as pltpu
from jax.experimental.pallas import tpu_sc as plsc

N = 10000
NP = 10240
E = 320000
C = 128
H = 64
NSUB = 16
NCORE = 2
ROW = 128
EP = 2560
E2 = EP * ROW
ECH = EP // NSUB
NPS = NP // NSUB

_f32 = jnp.float32


@functools.lru_cache(maxsize=None)
def _make_sc_layer():
  out_type = [
      jax.ShapeDtypeStruct((NCORE, NP, H), _f32),
      jax.ShapeDtypeStruct((NCORE, NP, H), _f32),
      jax.ShapeDtypeStruct((NP,), _f32),
      jax.ShapeDtypeStruct((NP,), _f32),
  ]
  mesh = plsc.VectorSubcoreMesh(core_axis_name="c", subcore_axis_name="s",
                                num_cores=NCORE, num_subcores=NSUB)
  scratch_types = [
      pltpu.VMEM_SHARED((NP, H), _f32),
      pltpu.VMEM_SHARED((NP,), _f32),
      pltpu.VMEM((4, 3, ROW), jnp.int32),
      pltpu.VMEM((3, ROW, H), _f32),
      pltpu.VMEM((ROW,), _f32),
      pltpu.VMEM((NPS, H), _f32),
      pltpu.VMEM((NPS,), _f32),
      pltpu.SemaphoreType.DMA((4,)),
      pltpu.SemaphoreType.DMA((3,)),
      pltpu.SemaphoreType.DMA((3,)),
      pltpu.SemaphoreType.DMA((3,)),
  ]

  def body(xg1, xg2, comb, agg1, agg2, deg1, deg2, *refs):
    (acc_sp, deg_sp, cb, rows, ones, zacc, zdeg,
     sem_i, sem_g, sem_s, sem_d) = refs
    c = lax.axis_index("c")
    s = lax.axis_index("s")
    rbase = s * NPS
    ebase = s * ECH

    @pl.loop(0, ROW // 16)
    def _(j):
      ones[pl.ds(j * 16, 16)] = jnp.ones((16,), _f32)

    @pl.loop(0, NPS // 16)
    def _(j):
      zdeg[pl.ds(j * 16, 16)] = jnp.zeros((16,), _f32)

    @pl.loop(0, NPS)
    def _(i):
      for b in range(H // 16):
        zacc[i, pl.ds(b * 16, 16)] = jnp.zeros((16,), _f32)

    def run_pass(xg, g_pl, s_pl, agg_out, deg_out, deg_core):
      pltpu.sync_copy(zacc, acc_sp.at[pl.ds(rbase, NPS)])

      @pl.when(c == deg_core)
      def _():
        pltpu.sync_copy(zdeg, deg_sp.at[pl.ds(rbase, NPS)])

      plsc.subcore_barrier()

      def idx_load(j):
        slot = lax.rem(j, 4)
        pltpu.async_copy(comb.at[ebase + j], cb.at[slot], sem_i.at[slot])

      def idx_wait(j):
        slot = lax.rem(j, 4)
        pltpu.make_async_copy(comb.at[ebase + j], cb.at[slot],
                              sem_i.at[slot]).wait()

      def prep(j):
        slot = lax.rem(j, 4)
        for q in range(ROW // 16):
          sl = pl.ds(q * 16, 16)
          cb[slot, g_pl, sl] = cb[slot, g_pl, sl] + c * NP

      def gather_issue(j):
        slot = lax.rem(j, 4)
        m = lax.rem(j, 3)
        pltpu.async_copy(xg.at[cb.at[slot, g_pl]], rows.at[m], sem_g.at[m])

      def gather_wait(j):
        slot = lax.rem(j, 4)
        m = lax.rem(j, 3)
        pltpu.make_async_copy(xg.at[cb.at[slot, g_pl]], rows.at[m],
                              sem_g.at[m]).wait()

      def scale(j):
        slot = lax.rem(j, 4)
        m = lax.rem(j, 3)

        @pl.loop(0, ROW // 16)
        def _(g):
          w16 = plsc.bitcast(cb[slot, 2, pl.ds(g * 16, 16)], _f32)
          base = g * 16
          for half in range(2):
            es = [base + half * 8 + e for e in range(8)]
            ws = [w16[half * 8 + e] for e in range(8)]
            sls = [pl.ds(b * 16, 16) for b in range(H // 16)]
            vals = [[rows[m, i, sl] for sl in sls] for i in es]
            for ei, i in enumerate(es):
              for b, sl in enumerate(sls):
                rows[m, i, sl] = vals[ei][b] * ws[ei]

      def scatter_issue(j):
        slot = lax.rem(j, 4)
        m = lax.rem(j, 3)
        pltpu.async_copy(rows.at[m], acc_sp.at[cb.at[slot, s_pl]],
                         sem_s.at[m], add=True)

        @pl.when(c == deg_core)
        def _():
          pltpu.async_copy(ones, deg_sp.at[cb.at[slot, s_pl]], sem_d.at[m],
                           add=True)

      def scatter_wait(j):
        slot = lax.rem(j, 4)
        m = lax.rem(j, 3)
        pltpu.make_async_copy(rows.at[m], acc_sp.at[cb.at[slot, s_pl]],
                              sem_s.at[m]).wait()

        @pl.when(c == deg_core)
        def _():
          pltpu.make_async_copy(ones, deg_sp.at[cb.at[slot, s_pl]],
                                sem_d.at[m]).wait()

      idx_load(0)
      idx_load(1)
      idx_load(2)
      idx_wait(0)
      prep(0)
      gather_issue(0)
      idx_wait(1)
      prep(1)
      gather_issue(1)

      @pl.loop(0, ECH)
      def _(j):
        j2 = j + 2

        @pl.when(j2 < ECH)
        def _():
          idx_wait(j2)
          prep(j2)

          @pl.when(j >= 1)
          def _():
            scatter_wait(j - 1)

          gather_issue(j2)

        gather_wait(j)
        scale(j)
        scatter_issue(j)

        @pl.when(j + 3 < ECH)
        def _():
          idx_load(j + 3)

      scatter_wait(ECH - 3)
      scatter_wait(ECH - 2)
      scatter_wait(ECH - 1)
      plsc.subcore_barrier()

      pltpu.sync_copy(acc_sp.at[pl.ds(rbase, NPS)],
                      agg_out.at[c, pl.ds(rbase, NPS)])

      @pl.when(c == deg_core)
      def _():
        pltpu.sync_copy(deg_sp.at[pl.ds(rbase, NPS)],
                        deg_out.at[pl.ds(rbase, NPS)])

      plsc.subcore_barrier()

    run_pass(xg1, 0, 1, agg1, deg1, 0)
    run_pass(xg2, 1, 0, agg2, deg2, 1)

  return pl.kernel(body, out_type=out_type, mesh=mesh,
                   scratch_types=scratch_types,
                   compiler_params=pltpu.CompilerParams(
                       use_tc_tiling_on_sc=False,
                       needs_layout_passes=False))


def _make_dense(relu, split_out):
  R = 1024
  grid = (NP // R,)

  def body(aggA, degA, xA, aggB, degB, xB,
           WlA, WrA, WlB, WrB, bA, bB, outA, outB):
    def half(agg, deg, x, Wl, Wr, b):
      inv = 1.0 / jnp.maximum(deg[...], 1.0)
      acc = jnp.dot(agg[0] * inv, Wl[0], preferred_element_type=_f32)
      acc += jnp.dot(agg[1] * inv, Wl[1], preferred_element_type=_f32)
      acc += jnp.dot(x[0], Wr[0], preferred_element_type=_f32)
      acc += jnp.dot(x[1], Wr[1], preferred_element_type=_f32)
      acc += b[...]
      if relu:
        acc = jnp.maximum(acc, 0.0)
      return acc

    a = half(aggA, degA, xA, WlA, WrA, bA)
    bb = half(aggB, degB, xB, WlB, WrB, bB)
    if split_out:
      outA[0], outA[1] = a[:, :H], a[:, H:]
      outB[0], outB[1] = bb[:, :H], bb[:, H:]
    else:
      outA[...] = a
      outB[...] = bb

  feat = pl.BlockSpec((NCORE, R, H), lambda i: (0, i, 0))
  deg = pl.BlockSpec((R, 1), lambda i: (i, 0))
  wspec = pl.BlockSpec((NCORE, H, C), lambda i: (0, 0, 0))
  bspec = pl.BlockSpec((1, C), lambda i: (0, 0))
  if split_out:
    out_specs = [feat, feat]
    out_shape = [jax.ShapeDtypeStruct((NCORE, NP, H), _f32)] * 2
  else:
    out_specs = [pl.BlockSpec((R, C), lambda i: (i, 0))] * 2
    out_shape = [jax.ShapeDtypeStruct((NP, C), _f32)] * 2
  return pl.pallas_call(
      body,
      grid=grid,
      in_specs=[feat, deg, feat, feat, deg, feat,
                wspec, wspec, wspec, wspec, bspec, bspec],
      out_specs=out_specs,
      out_shape=out_shape,
  )


_dense0 = _make_dense(relu=True, split_out=True)
_dense1 = _make_dense(relu=False, split_out=False)


def _sig_body(x_ref, o_ref):
  o_ref[...] = jax.nn.sigmoid(x_ref[...])


_sigmoid_tc = pl.pallas_call(
    _sig_body, out_shape=jax.ShapeDtypeStruct((EP, ROW), _f32))


def _to_halves(x):
  xp = jnp.pad(x, ((0, NP - N), (0, 0)))
  return xp.reshape(NP, NCORE, H).transpose(1, 0, 2)

@jax.jit
def kernel(s, t, edge_index, edge_weight,
           Ws_l0, Ws_r0, Wt_l0, Wt_r0,
           Ws_l1, Ws_r1, Wt_l1, Wt_r1,
           bs0, bt0, bs1, bt1):
  s2 = _to_halves(s)
  t2 = _to_halves(t)
  pad = E2 - E
  pad_idx = N + jnp.arange(pad, dtype=jnp.int32) % (NP - N)
  src = jnp.concatenate([edge_index[0], pad_idx])
  dst = jnp.concatenate([edge_index[1], pad_idx])
  ewp = jnp.concatenate([edge_weight, jnp.zeros((pad,), _f32)])
  sig = _sigmoid_tc(ewp.reshape(EP, ROW))
  comb = jnp.stack([src.reshape(EP, ROW), dst.reshape(EP, ROW),
                    jax.lax.bitcast_convert_type(sig, jnp.int32)], axis=1)

  flat = lambda x: x.reshape(NCORE * NP, H)
  aggs0, aggt0, degd, degs = _make_sc_layer()(flat(t2), flat(s2), comb)
  degd = degd.reshape(NP, 1)
  degs = degs.reshape(NP, 1)

  wl = lambda w: w.reshape(NCORE, H, C)
  wr = wl
  s1, t1 = _dense0(aggs0, degd, t2, aggt0, degs, s2,
                   wl(Ws_l0), wr(Ws_r0), wl(Wt_l0), wr(Wt_r0),
                   bs0.reshape(1, C), bt0.reshape(1, C))

  aggs1, aggt1, _, _ = _make_sc_layer()(flat(t1), flat(s1), comb)
  s_out, t_out = _dense1(aggs1, degd, t1, aggt1, degs, s1,
                         wl(Ws_l1), wr(Ws_r1), wl(Wt_l1), wr(Wt_r1),
                         bs1.reshape(1, C), bt1.reshape(1, C))
  return (s_out[:N], t_out[:N])


# speedup vs baseline: 1.2354x; 1.1539x over previous
import functools

import jax
import jax.numpy as jnp
from jax import lax
from jax.experimental import pallas as pl
from jax.experimental.pallas import tpu as pltpu
from jax.experimental.pallas import tpu_sc as plsc

N = 10000
NP = 10240
E = 320000
C = 128
H = 64
NSUB = 16
NCORE = 2
ROW = 128
EP = 2560
E2 = EP * ROW
ECH = EP // NSUB
NPS = NP // NSUB

_f32 = jnp.float32


@functools.lru_cache(maxsize=None)
def _make_sc_layer():
  out_type = [
      jax.ShapeDtypeStruct((NCORE, NP, H), _f32),
      jax.ShapeDtypeStruct((NCORE, NP, H), _f32),
      jax.ShapeDtypeStruct((NP,), _f32),
      jax.ShapeDtypeStruct((NP,), _f32),
  ]
  mesh = plsc.VectorSubcoreMesh(core_axis_name="c", subcore_axis_name="s",
                                num_cores=NCORE, num_subcores=NSUB)
  scratch_types = [
      pltpu.VMEM_SHARED((NP, H), _f32),
      pltpu.VMEM_SHARED((NP,), _f32),
      pltpu.VMEM((4, 3, ROW), jnp.int32),
      pltpu.VMEM((3, ROW, H), _f32),
      pltpu.VMEM((ROW,), _f32),
      pltpu.VMEM((NPS, H), _f32),
      pltpu.VMEM((NPS,), _f32),
      pltpu.SemaphoreType.DMA((4,)),
      pltpu.SemaphoreType.DMA((3,)),
      pltpu.SemaphoreType.DMA((3,)),
      pltpu.SemaphoreType.DMA((3,)),
  ]

  def body(xg1, xg2, comb, agg1, agg2, deg1, deg2, *refs):
    (acc_sp, deg_sp, cb, rows, ones, zacc, zdeg,
     sem_i, sem_g, sem_s, sem_d) = refs
    c = lax.axis_index("c")
    s = lax.axis_index("s")
    rbase = s * NPS
    ebase = s * ECH

    @pl.loop(0, ROW // 16)
    def _(j):
      ones[pl.ds(j * 16, 16)] = jnp.ones((16,), _f32)

    @pl.loop(0, NPS // 16)
    def _(j):
      zdeg[pl.ds(j * 16, 16)] = jnp.zeros((16,), _f32)

    @pl.loop(0, NPS)
    def _(i):
      for b in range(H // 16):
        zacc[i, pl.ds(b * 16, 16)] = jnp.zeros((16,), _f32)

    def run_pass(xg, g_pl, s_pl, agg_out, deg_out, deg_core):
      pltpu.sync_copy(zacc, acc_sp.at[pl.ds(rbase, NPS)])

      @pl.when(c == deg_core)
      def _():
        pltpu.sync_copy(zdeg, deg_sp.at[pl.ds(rbase, NPS)])

      plsc.subcore_barrier()

      def idx_load(j):
        slot = lax.rem(j, 4)
        pltpu.async_copy(comb.at[ebase + j], cb.at[slot], sem_i.at[slot])

      def idx_wait(j):
        slot = lax.rem(j, 4)
        pltpu.make_async_copy(comb.at[ebase + j], cb.at[slot],
                              sem_i.at[slot]).wait()

      def prep(j):
        slot = lax.rem(j, 4)
        for q in range(ROW // 16):
          sl = pl.ds(q * 16, 16)
          cb[slot, g_pl, sl] = cb[slot, g_pl, sl] + c * NP

      def gather_issue(j):
        slot = lax.rem(j, 4)
        m = lax.rem(j, 3)
        pltpu.async_copy(xg.at[cb.at[slot, g_pl]], rows.at[m], sem_g.at[m])

      def gather_wait(j):
        slot = lax.rem(j, 4)
        m = lax.rem(j, 3)
        pltpu.make_async_copy(xg.at[cb.at[slot, g_pl]], rows.at[m],
                              sem_g.at[m]).wait()

      def scale(j):
        slot = lax.rem(j, 4)
        m = lax.rem(j, 3)

        @pl.loop(0, ROW // 16)
        def _(g):
          w16 = plsc.bitcast(cb[slot, 2, pl.ds(g * 16, 16)], _f32)
          base = g * 16
          for half in range(2):
            es = [base + half * 8 + e for e in range(8)]
            ws = [w16[half * 8 + e] for e in range(8)]
            sls = [pl.ds(b * 16, 16) for b in range(H // 16)]
            vals = [[rows[m, i, sl] for sl in sls] for i in es]
            for ei, i in enumerate(es):
              for b, sl in enumerate(sls):
                rows[m, i, sl] = vals[ei][b] * ws[ei]

      def scatter_issue(j):
        slot = lax.rem(j, 4)
        m = lax.rem(j, 3)
        pltpu.async_copy(rows.at[m], acc_sp.at[cb.at[slot, s_pl]],
                         sem_s.at[m], add=True)

        @pl.when(c == deg_core)
        def _():
          pltpu.async_copy(ones, deg_sp.at[cb.at[slot, s_pl]], sem_d.at[m],
                           add=True)

      def scatter_wait(j):
        slot = lax.rem(j, 4)
        m = lax.rem(j, 3)
        pltpu.make_async_copy(rows.at[m], acc_sp.at[cb.at[slot, s_pl]],
                              sem_s.at[m]).wait()

        @pl.when(c == deg_core)
        def _():
          pltpu.make_async_copy(ones, deg_sp.at[cb.at[slot, s_pl]],
                                sem_d.at[m]).wait()

      idx_load(0)
      idx_load(1)
      idx_load(2)
      idx_wait(0)
      prep(0)
      gather_issue(0)
      idx_wait(1)
      prep(1)
      gather_issue(1)

      @pl.loop(0, ECH)
      def _(j):
        j2 = j + 2
        gather_wait(j)
        scale(j)
        scatter_issue(j)

        @pl.when(j2 < ECH)
        def _():
          idx_wait(j2)
          prep(j2)

          @pl.when(j >= 1)
          def _():
            scatter_wait(j - 1)

          gather_issue(j2)

        @pl.when(j + 3 < ECH)
        def _():
          idx_load(j + 3)

      scatter_wait(ECH - 3)
      scatter_wait(ECH - 2)
      scatter_wait(ECH - 1)
      plsc.subcore_barrier()

      pltpu.sync_copy(acc_sp.at[pl.ds(rbase, NPS)],
                      agg_out.at[c, pl.ds(rbase, NPS)])

      @pl.when(c == deg_core)
      def _():
        pltpu.sync_copy(deg_sp.at[pl.ds(rbase, NPS)],
                        deg_out.at[pl.ds(rbase, NPS)])

      plsc.subcore_barrier()

    run_pass(xg1, 0, 1, agg1, deg1, 0)
    run_pass(xg2, 1, 0, agg2, deg2, 1)

  return pl.kernel(body, out_type=out_type, mesh=mesh,
                   scratch_types=scratch_types,
                   compiler_params=pltpu.CompilerParams(
                       use_tc_tiling_on_sc=False,
                       needs_layout_passes=False))


def _make_dense(relu, split_out):
  R = 1024
  grid = (NP // R,)

  def body(aggA, degA, xA, aggB, degB, xB,
           WlA, WrA, WlB, WrB, bA, bB, outA, outB):
    def half(agg, deg, x, Wl, Wr, b):
      inv = 1.0 / jnp.maximum(deg[...], 1.0)
      acc = jnp.dot(agg[0] * inv, Wl[0], preferred_element_type=_f32)
      acc += jnp.dot(agg[1] * inv, Wl[1], preferred_element_type=_f32)
      acc += jnp.dot(x[0], Wr[0], preferred_element_type=_f32)
      acc += jnp.dot(x[1], Wr[1], preferred_element_type=_f32)
      acc += b[...]
      if relu:
        acc = jnp.maximum(acc, 0.0)
      return acc

    a = half(aggA, degA, xA, WlA, WrA, bA)
    bb = half(aggB, degB, xB, WlB, WrB, bB)
    if split_out:
      outA[0], outA[1] = a[:, :H], a[:, H:]
      outB[0], outB[1] = bb[:, :H], bb[:, H:]
    else:
      outA[...] = a
      outB[...] = bb

  feat = pl.BlockSpec((NCORE, R, H), lambda i: (0, i, 0))
  deg = pl.BlockSpec((R, 1), lambda i: (i, 0))
  wspec = pl.BlockSpec((NCORE, H, C), lambda i: (0, 0, 0))
  bspec = pl.BlockSpec((1, C), lambda i: (0, 0))
  if split_out:
    out_specs = [feat, feat]
    out_shape = [jax.ShapeDtypeStruct((NCORE, NP, H), _f32)] * 2
  else:
    out_specs = [pl.BlockSpec((R, C), lambda i: (i, 0))] * 2
    out_shape = [jax.ShapeDtypeStruct((NP, C), _f32)] * 2
  return pl.pallas_call(
      body,
      grid=grid,
      in_specs=[feat, deg, feat, feat, deg, feat,
                wspec, wspec, wspec, wspec, bspec, bspec],
      out_specs=out_specs,
      out_shape=out_shape,
  )


_dense0 = _make_dense(relu=True, split_out=True)
_dense1 = _make_dense(relu=False, split_out=False)


def _sig_body(x_ref, o_ref):
  o_ref[...] = jax.nn.sigmoid(x_ref[...])


_sigmoid_tc = pl.pallas_call(
    _sig_body, out_shape=jax.ShapeDtypeStruct((EP, ROW), _f32))


def _to_halves(x):
  xp = jnp.pad(x, ((0, NP - N), (0, 0)))
  return xp.reshape(NP, NCORE, H).transpose(1, 0, 2)

@jax.jit
def kernel(s, t, edge_index, edge_weight,
           Ws_l0, Ws_r0, Wt_l0, Wt_r0,
           Ws_l1, Ws_r1, Wt_l1, Wt_r1,
           bs0, bt0, bs1, bt1):
  s2 = _to_halves(s)
  t2 = _to_halves(t)
  pad = E2 - E
  pad_idx = N + jnp.arange(pad, dtype=jnp.int32) % (NP - N)
  src = jnp.concatenate([edge_index[0], pad_idx])
  dst = jnp.concatenate([edge_index[1], pad_idx])
  ewp = jnp.concatenate([edge_weight, jnp.zeros((pad,), _f32)])
  sig = _sigmoid_tc(ewp.reshape(EP, ROW))
  comb = jnp.stack([src.reshape(EP, ROW), dst.reshape(EP, ROW),
                    jax.lax.bitcast_convert_type(sig, jnp.int32)], axis=1)

  flat = lambda x: x.reshape(NCORE * NP, H)
  aggs0, aggt0, degd, degs = _make_sc_layer()(flat(t2), flat(s2), comb)
  degd = degd.reshape(NP, 1)
  degs = degs.reshape(NP, 1)

  wl = lambda w: w.reshape(NCORE, H, C)
  wr = wl
  s1, t1 = _dense0(aggs0, degd, t2, aggt0, degs, s2,
                   wl(Ws_l0), wr(Ws_r0), wl(Wt_l0), wr(Wt_r0),
                   bs0.reshape(1, C), bt0.reshape(1, C))

  aggs1, aggt1, _, _ = _make_sc_layer()(flat(t1), flat(s1), comb)
  s_out, t_out = _dense1(aggs1, degd, t1, aggt1, degs, s1,
                         wl(Ws_l1), wr(Ws_r1), wl(Wt_l1), wr(Wt_r1),
                         bs1.reshape(1, C), bt1.reshape(1, C))
  return (s_out[:N], t_out[:N])
